# baseline (speedup 1.0000x reference)
import functools

import jax
import jax.numpy as jnp
from jax import lax
from jax.experimental import pallas as pl
from jax.experimental.pallas import tpu as pltpu
from jax.experimental.pallas import tpu_sc as plsc

_NE = 8
_NN = 10000
_DF = 128
_EDGES = 320000
_DC = _NE * _DF

_NC = 2
_NS = 16
_NW = _NC * _NS
_EPW = _EDGES // _NW
_CH = 16
_NCH = _EPW // _CH
_L = 16
_HIMASK = -65536
_INV_SCALE = float(2.0 ** -62)


def _tc_normalize(z, sqrt_c):
    nb = 400

    def body(c_ref, z_ref, o_ref):
        x = z_ref[0]
        n = jnp.sqrt(jnp.sum(x * x, axis=1, keepdims=True))
        t = x / jnp.maximum(n, 1e-12) * c_ref[0]
        o_ref[...] = jnp.round(t * 32768.0).astype(jnp.int16)

    return pl.pallas_call(
        body,
        grid=(_NE, _NN // nb),
        in_specs=[
            pl.BlockSpec((1, 1, _DF), lambda k, n: (k, 0, 0)),
            pl.BlockSpec((1, nb, _DF), lambda k, n: (k, n, 0)),
        ],
        out_specs=pl.BlockSpec((nb, _DF), lambda k, n: (n, k)),
        out_shape=jax.ShapeDtypeStruct((_NN, _DC), jnp.int16),
    )(sqrt_c, z)


def _sc_gather_dot(table, src, dst):
    mesh = plsc.VectorSubcoreMesh(core_axis_name="c", subcore_axis_name="s")

    @functools.partial(
        pl.kernel,
        mesh=mesh,
        out_type=jax.ShapeDtypeStruct((_EDGES,), jnp.float32),
        scratch_types=[
            pltpu.VMEM((_EPW,), jnp.int32),
            pltpu.VMEM((_EPW,), jnp.int32),
            pltpu.VMEM((_EPW,), jnp.float32),
            pltpu.VMEM((_CH, _DC // 2), jnp.int32),
            pltpu.VMEM((_CH, _DC // 2), jnp.int32),
            pltpu.VMEM((_CH, _DC // 2), jnp.int32),
            pltpu.VMEM((_CH, _DC // 2), jnp.int32),
            pltpu.VMEM((_CH, _DC // 2), jnp.int32),
            pltpu.VMEM((_CH, _DC // 2), jnp.int32),
            pltpu.VMEM((_CH, _DC // 2), jnp.int32),
            pltpu.VMEM((_CH, _DC // 2), jnp.int32),
            pltpu.SemaphoreType.DMA,
            pltpu.SemaphoreType.DMA,
            pltpu.SemaphoreType.DMA,
            pltpu.SemaphoreType.DMA,
        ],
    )
    def k(table_hbm, src_hbm, dst_hbm, out_hbm,
          src_v, dst_v, out_v, srows0, drows0, srows1, drows1,
          srows2, drows2, srows3, drows3, sem0, sem1, sem2, sem3):
        wid = lax.axis_index("s") * _NC + lax.axis_index("c")
        base = pl.multiple_of(wid * _EPW, 8)
        pltpu.sync_copy(src_hbm.at[pl.ds(base, _EPW)], src_v)
        pltpu.sync_copy(dst_hbm.at[pl.ds(base, _EPW)], dst_v)

        lanes = jnp.arange(_L, dtype=jnp.int32)
        perms = [lanes ^ s for s in (8, 4, 2, 1)]
        def start(g, sbuf, dbuf, sem):
            off = pl.multiple_of(g * _CH, _CH)
            pltpu.async_copy(table_hbm.at[src_v[pl.ds(off, _CH)]], sbuf, sem)
            pltpu.async_copy(table_hbm.at[dst_v[pl.ds(off, _CH)]], dbuf, sem)

        def wait(sbuf, dbuf, sem):
            pltpu.make_async_copy(table_hbm.at[pl.ds(0, _CH)], sbuf, sem).wait()
            pltpu.make_async_copy(table_hbm.at[pl.ds(0, _CH)], dbuf, sem).wait()

        def compute(g, sbuf, dbuf):
            def edge_body(e, res):
                acc = jnp.zeros((_L,), jnp.float32)
                for j in range(_DC // 32):
                    us = sbuf[e, pl.ds(j * _L, _L)]
                    ud = dbuf[e, pl.ds(j * _L, _L)]
                    sh = lax.convert_element_type(us & _HIMASK, jnp.float32)
                    dh = lax.convert_element_type(ud & _HIMASK, jnp.float32)
                    sl = lax.convert_element_type(us << 16, jnp.float32)
                    dl = lax.convert_element_type(ud << 16, jnp.float32)
                    acc = acc + sh * dh + sl * dl
                for p in perms:
                    acc = acc + acc.at[p].get(mode="promise_in_bounds")
                return jnp.where(lanes == e, acc, res)

            res = lax.fori_loop(0, _CH, edge_body,
                                jnp.zeros((_L,), jnp.float32))
            off = pl.multiple_of(g * _CH, _CH)
            out_v[pl.ds(off, _CH)] = jnp.clip(res * _INV_SCALE, 0.0, 1.0)

        bufs = ((srows0, drows0, sem0),
                (srows1, drows1, sem1),
                (srows2, drows2, sem2),
                (srows3, drows3, sem3))
        for b in range(4):
            start(b, *bufs[b])

        def ring_body(g, carry):
            c0 = 4 * g
            for b in range(4):
                sbuf, dbuf, sem = bufs[b]
                wait(sbuf, dbuf, sem)
                compute(c0 + b, sbuf, dbuf)

                @pl.when(c0 + b + 4 < _NCH)
                def _():
                    start(c0 + b + 4, sbuf, dbuf, sem)
            return carry

        lax.fori_loop(0, (_NCH - 1) // 4, ring_body, 0)
        wait(*bufs[0])
        compute(_NCH - 1, bufs[0][0], bufs[0][1])
        pltpu.sync_copy(out_v, out_hbm.at[pl.ds(base, _EPW)])

    return k(table, src, dst)


def kernel(z, z_conc, edge_index, noise_scale, tau, num_experts):
    del z_conc, noise_scale
    betas = jnp.linspace(1e-4, 0.02, _NE)
    alphas_cumprod = jnp.cumprod(1.0 - betas)
    sqrt_one_minus = jnp.sqrt(1.0 - alphas_cumprod)
    cumulative_blend = jnp.flip(jnp.cumsum(jnp.flip(sqrt_one_minus)))
    tau_c = jnp.clip(jnp.asarray(tau, jnp.int32), 1, _NE)
    ne_c = jnp.clip(jnp.asarray(num_experts, jnp.int32), 1, _NE)
    nf = jnp.take(cumulative_blend, tau_c - 1)
    ls = jnp.arange(1, _NE + 1)
    inc = jnp.logical_and(ls >= tau_c, ls <= ne_c).astype(jnp.float32)
    c = sqrt_one_minus * inc / nf
    sqrt_c = jnp.broadcast_to(jnp.sqrt(c)[:, None, None], (_NE, 1, _DF))

    table = _tc_normalize(z, sqrt_c)
    table_i32 = lax.bitcast_convert_type(
        table.reshape(_NN, _DC // 2, 2), jnp.int32)
    ei = edge_index.astype(jnp.int32)
    return _sc_gather_dot(table_i32, ei[0], ei[1])


# speedup vs baseline: 1.4122x; 1.4122x over previous
import functools

import jax
import jax.numpy as jnp
from jax import lax
from jax.experimental import pallas as pl
from jax.experimental.pallas import tpu as pltpu
from jax.experimental.pallas import tpu_sc as plsc

_NE = 8
_NN = 10000
_DF = 128
_EDGES = 320000
_DC = _NE * _DF

_NC = 2
_NS = 16
_NW = _NC * _NS
_EPW = _EDGES // _NW
_CH = 16
_NCH = _EPW // _CH
_L = 16


def _tc_normalize(z, sqrt_c):
    nb = 400

    def body(c_ref, z_ref, o_ref):
        x = z_ref[0]
        n = jnp.sqrt(jnp.sum(x * x, axis=1, keepdims=True))
        o_ref[...] = x / jnp.maximum(n, 1e-12) * c_ref[0]

    return pl.pallas_call(
        body,
        grid=(_NE, _NN // nb),
        in_specs=[
            pl.BlockSpec((1, 1, _DF), lambda k, n: (k, 0, 0)),
            pl.BlockSpec((1, nb, _DF), lambda k, n: (k, n, 0)),
        ],
        out_specs=pl.BlockSpec((nb, _DF), lambda k, n: (n, k)),
        out_shape=jax.ShapeDtypeStruct((_NN, _DC), jnp.float32),
    )(sqrt_c, z)


def _sc_gather_dot(table, src, dst):
    mesh = plsc.VectorSubcoreMesh(core_axis_name="c", subcore_axis_name="s")

    @functools.partial(
        pl.kernel,
        mesh=mesh,
        out_type=jax.ShapeDtypeStruct((_EDGES,), jnp.float32),
        scratch_types=[
            pltpu.VMEM((_EPW,), jnp.int32),
            pltpu.VMEM((_EPW,), jnp.int32),
            pltpu.VMEM((_EPW,), jnp.float32),
            pltpu.VMEM((_CH, _DC), jnp.float32),
            pltpu.VMEM((_CH, _DC), jnp.float32),
            pltpu.VMEM((_CH, _DC), jnp.float32),
            pltpu.VMEM((_CH, _DC), jnp.float32),
            pltpu.VMEM((_CH, _DC), jnp.float32),
            pltpu.VMEM((_CH, _DC), jnp.float32),
            pltpu.SemaphoreType.DMA,
            pltpu.SemaphoreType.DMA,
            pltpu.SemaphoreType.DMA,
        ],
    )
    def k(table_hbm, src_hbm, dst_hbm, out_hbm,
          src_v, dst_v, out_v, srows0, drows0, srows1, drows1,
          srows2, drows2, sem0, sem1, sem2):
        wid = lax.axis_index("s") * _NC + lax.axis_index("c")
        base = pl.multiple_of(wid * _EPW, 8)
        pltpu.sync_copy(src_hbm.at[pl.ds(base, _EPW)], src_v)
        pltpu.sync_copy(dst_hbm.at[pl.ds(base, _EPW)], dst_v)

        lanes = jnp.arange(_L, dtype=jnp.int32)
        perms = [lanes ^ s for s in (8, 4, 2, 1)]
        def start(g, sbuf, dbuf, sem):
            off = pl.multiple_of(g * _CH, _CH)
            pltpu.async_copy(table_hbm.at[src_v[pl.ds(off, _CH)]], sbuf, sem)
            pltpu.async_copy(table_hbm.at[dst_v[pl.ds(off, _CH)]], dbuf, sem)

        def wait(sbuf, dbuf, sem):
            pltpu.make_async_copy(table_hbm.at[pl.ds(0, _CH)], sbuf, sem).wait()
            pltpu.make_async_copy(table_hbm.at[pl.ds(0, _CH)], dbuf, sem).wait()

        def compute(g, sbuf, dbuf):
            def edge_body(e, res):
                acc = sbuf[e, pl.ds(0, _L)] * dbuf[e, pl.ds(0, _L)]
                for j in range(1, _DC // _L):
                    acc = acc + (sbuf[e, pl.ds(j * _L, _L)] *
                                 dbuf[e, pl.ds(j * _L, _L)])
                for p in perms:
                    acc = acc + acc.at[p].get(mode="promise_in_bounds")
                return jnp.where(lanes == e, acc, res)

            res = lax.fori_loop(0, _CH, edge_body,
                                jnp.zeros((_L,), jnp.float32))
            off = pl.multiple_of(g * _CH, _CH)
            out_v[pl.ds(off, _CH)] = jnp.clip(res, 0.0, 1.0)

        bufs = ((srows0, drows0, sem0),
                (srows1, drows1, sem1),
                (srows2, drows2, sem2))
        for b in range(3):
            start(b, *bufs[b])

        def ring_body(g, carry):
            c0 = 3 * g
            for b in range(3):
                sbuf, dbuf, sem = bufs[b]
                wait(sbuf, dbuf, sem)
                compute(c0 + b, sbuf, dbuf)

                @pl.when(c0 + b + 3 < _NCH)
                def _():
                    start(c0 + b + 3, sbuf, dbuf, sem)
            return carry

        lax.fori_loop(0, (_NCH - 1) // 3, ring_body, 0)
        wait(*bufs[0])
        compute(_NCH - 1, bufs[0][0], bufs[0][1])
        pltpu.sync_copy(out_v, out_hbm.at[pl.ds(base, _EPW)])

    return k(table, src, dst)


def kernel(z, z_conc, edge_index, noise_scale, tau, num_experts):
    del z_conc, noise_scale
    betas = jnp.linspace(1e-4, 0.02, _NE)
    alphas_cumprod = jnp.cumprod(1.0 - betas)
    sqrt_one_minus = jnp.sqrt(1.0 - alphas_cumprod)
    cumulative_blend = jnp.flip(jnp.cumsum(jnp.flip(sqrt_one_minus)))
    tau_c = jnp.clip(jnp.asarray(tau, jnp.int32), 1, _NE)
    ne_c = jnp.clip(jnp.asarray(num_experts, jnp.int32), 1, _NE)
    nf = jnp.take(cumulative_blend, tau_c - 1)
    ls = jnp.arange(1, _NE + 1)
    inc = jnp.logical_and(ls >= tau_c, ls <= ne_c).astype(jnp.float32)
    c = sqrt_one_minus * inc / nf
    sqrt_c = jnp.broadcast_to(jnp.sqrt(c)[:, None, None], (_NE, 1, _DF))

    table = _tc_normalize(z, sqrt_c)
    ei = edge_index.astype(jnp.int32)
    return _sc_gather_dot(table, ei[0], ei[1])
